# baseline (speedup 1.0000x reference)
import functools

import jax
import jax.numpy as jnp
from jax import lax
from jax.experimental import pallas as pl
from jax.experimental.pallas import tpu as pltpu
from jax.experimental.pallas import tpu_sc as plsc

NC = 2
NS = 16
CL = 20


def kernel(feat, spatial_emb, temporal_emb):
    B, L, O, D = feat.shape
    NK = D // 16
    mesh = plsc.VectorSubcoreMesh(core_axis_name="c", subcore_axis_name="s")

    @functools.partial(
        pl.kernel,
        mesh=mesh,
        out_type=jax.ShapeDtypeStruct((B, L, O, D), jnp.float32),
        scratch_types=[
            pltpu.VMEM((CL, O, D), jnp.float32),
            pltpu.VMEM((L, D), jnp.float32),
            pltpu.VMEM((O, D), jnp.float32),
            pltpu.SemaphoreType.DMA,
        ],
    )
    def k(t_hbm, s_hbm, f_hbm, o_hbm, buf, t_v, s_v, sem):
        wid = lax.axis_index("s") * NC + lax.axis_index("c")
        pltpu.sync_copy(t_hbm, t_v)
        pltpu.sync_copy(s_hbm, s_v)
        ncl = L // CL

        def chunk_body(ci, carry):
            b = (B // (NC * NS)) * wid + lax.div(ci, ncl)
            l0 = lax.rem(ci, ncl) * CL
            pltpu.async_copy(f_hbm.at[b, pl.ds(l0, CL)], buf, sem).wait()

            def l_body(l, c2):
                tv = [t_v[l0 + l, pl.ds(kk * 16, 16)] for kk in range(NK)]
                for o in range(O):
                    for kk in range(NK):
                        sl = pl.ds(kk * 16, 16)
                        buf[l, o, sl] = buf[l, o, sl] + (tv[kk] + s_v[o, sl])
                return c2

            lax.fori_loop(0, CL, l_body, 0)
            pltpu.async_copy(buf, o_hbm.at[b, pl.ds(l0, CL)], sem).wait()
            return carry

        lax.fori_loop(0, (B // (NC * NS)) * ncl, chunk_body, 0)

    return k(temporal_emb, spatial_emb, feat)
